# baseline (speedup 1.0000x reference)
import functools

import jax
import jax.numpy as jnp
from jax import lax
from jax.experimental import pallas as pl
from jax.experimental.pallas import tpu as pltpu
from jax.experimental.pallas import tpu_sc as plsc

D_MODEL = 1024
SEQ_LEN = 2048
TOTAL_ROWS = 8192
NUM_CORES = 2
NUM_SUBCORES = 16
NUM_WORKERS = NUM_CORES * NUM_SUBCORES
ROWS_PER_WORKER = TOTAL_ROWS // NUM_WORKERS
CHUNK = 32
NUM_CHUNKS = ROWS_PER_WORKER // CHUNK
LANES = 16
VECS_PER_ROW = D_MODEL // LANES


def _body(ids_hbm, wtab_hbm, ptab_hbm, out_hbm, idx_v, wbuf, pbuf, wsem, psem):
    wid = lax.axis_index("s") * NUM_CORES + lax.axis_index("c")
    base = wid * ROWS_PER_WORKER
    s0 = base % SEQ_LEN
    pltpu.sync_copy(ids_hbm.at[pl.ds(base, ROWS_PER_WORKER)], idx_v)

    def chunk_step(c, carry):
        off = c * CHUNK
        gw = pltpu.async_copy(wtab_hbm.at[idx_v.at[pl.ds(off, CHUNK)]], wbuf, wsem)
        gp = pltpu.async_copy(ptab_hbm.at[pl.ds(s0 + off, CHUNK)], pbuf, psem)
        gw.wait()
        gp.wait()

        def add_row(r, carry2):
            def add_vec(j, carry3):
                sl = pl.ds(j * LANES, LANES)
                wbuf[r, sl] = wbuf[r, sl] + pbuf[r, sl]
                return carry3

            return lax.fori_loop(0, VECS_PER_ROW, add_vec, carry2)

        lax.fori_loop(0, CHUNK, add_row, carry)
        pltpu.sync_copy(wbuf, out_hbm.at[pl.ds(base + off, CHUNK)])
        return carry

    lax.fori_loop(0, NUM_CHUNKS, chunk_step, 0)


_embed = pl.kernel(
    _body,
    out_type=jax.ShapeDtypeStruct((TOTAL_ROWS, D_MODEL), jnp.float32),
    mesh=plsc.VectorSubcoreMesh(core_axis_name="c", subcore_axis_name="s"),
    scratch_types=[
        pltpu.VMEM((ROWS_PER_WORKER,), jnp.int32),
        pltpu.VMEM((CHUNK, D_MODEL), jnp.float32),
        pltpu.VMEM((CHUNK, D_MODEL), jnp.float32),
        pltpu.SemaphoreType.DMA,
        pltpu.SemaphoreType.DMA,
    ],
)


@jax.jit
def kernel(input_ids, word_embedding, position_embedding):
    batch, seq = input_ids.shape
    ids = input_ids.reshape(-1).astype(jnp.int32)
    out = _embed(ids, word_embedding, position_embedding)
    return out.reshape(batch, seq, D_MODEL)


# speedup vs baseline: 2.5758x; 2.5758x over previous
import jax
import jax.numpy as jnp
from jax import lax
from jax.experimental import pallas as pl
from jax.experimental.pallas import tpu as pltpu
from jax.experimental.pallas import tpu_sc as plsc

D_MODEL = 1024
SEQ_LEN = 2048
BATCH = 4
TOTAL_ROWS = BATCH * SEQ_LEN
NUM_CORES = 2
NUM_SUBCORES = 16
NUM_WORKERS = NUM_CORES * NUM_SUBCORES
POS_PER_WORKER = SEQ_LEN // NUM_WORKERS
CHUNK_POS = 8
NUM_CHUNKS = POS_PER_WORKER // CHUNK_POS
LANES = 16


def _body(ids_hbm, wtab_hbm, ptab_hbm, out_hbm, idx_v, wbuf, pbuf,
          wsem0, wsem1, psem0, psem1, osem0, osem1):
    wsem = (wsem0, wsem1)
    psem = (psem0, psem1)
    osem = (osem0, osem1)
    wid = lax.axis_index("s") * NUM_CORES + lax.axis_index("c")
    p0 = wid * POS_PER_WORKER

    for b in range(BATCH):
        pltpu.sync_copy(ids_hbm.at[pl.ds(b * SEQ_LEN + p0, POS_PER_WORKER)],
                        idx_v.at[b])

    gathers = [None] * NUM_CHUNKS
    poscopies = [None] * NUM_CHUNKS
    stores = [None] * NUM_CHUNKS

    def issue(t):
        par = t % 2
        gathers[t] = [
            pltpu.async_copy(
                wtab_hbm.at[idx_v.at[b, pl.ds(t * CHUNK_POS, CHUNK_POS)]],
                wbuf.at[par, b], wsem[par])
            for b in range(BATCH)
        ]
        poscopies[t] = pltpu.async_copy(
            ptab_hbm.at[pl.ds(p0 + t * CHUNK_POS, CHUNK_POS)],
            pbuf.at[par], psem[par])

    issue(0)
    for t in range(NUM_CHUNKS):
        par = t % 2
        if t + 1 < NUM_CHUNKS:
            if t >= 1:
                for c in stores[t - 1]:
                    c.wait()
            issue(t + 1)
        for c in gathers[t]:
            c.wait()
        poscopies[t].wait()

        def add_pos(p, carry):
            @plsc.parallel_loop(0, D_MODEL, step=LANES, unroll=4)
            def add_vec(o):
                js = pl.ds(o, LANES)
                pv = pbuf[par, p, js]
                for b in range(BATCH):
                    wbuf[par, b, p, js] = wbuf[par, b, p, js] + pv

            return carry

        lax.fori_loop(0, CHUNK_POS, add_pos, 0)

        stores[t] = [
            pltpu.async_copy(
                wbuf.at[par, b],
                out_hbm.at[pl.ds(b * SEQ_LEN + p0 + t * CHUNK_POS, CHUNK_POS)],
                osem[par])
            for b in range(BATCH)
        ]

    for t in (NUM_CHUNKS - 2, NUM_CHUNKS - 1):
        for c in stores[t]:
            c.wait()


_embed = pl.kernel(
    _body,
    out_type=jax.ShapeDtypeStruct((TOTAL_ROWS, D_MODEL), jnp.float32),
    mesh=plsc.VectorSubcoreMesh(core_axis_name="c", subcore_axis_name="s"),
    scratch_types=[
        pltpu.VMEM((BATCH, POS_PER_WORKER), jnp.int32),
        pltpu.VMEM((2, BATCH, CHUNK_POS, D_MODEL), jnp.float32),
        pltpu.VMEM((2, CHUNK_POS, D_MODEL), jnp.float32),
        pltpu.SemaphoreType.DMA,
        pltpu.SemaphoreType.DMA,
        pltpu.SemaphoreType.DMA,
        pltpu.SemaphoreType.DMA,
        pltpu.SemaphoreType.DMA,
        pltpu.SemaphoreType.DMA,
    ],
)


@jax.jit
def kernel(input_ids, word_embedding, position_embedding):
    batch, seq = input_ids.shape
    ids = input_ids.reshape(-1).astype(jnp.int32)
    out = _embed(ids, word_embedding, position_embedding)
    return out.reshape(batch, seq, D_MODEL)


# speedup vs baseline: 2.6238x; 1.0186x over previous
import jax
import jax.numpy as jnp
from jax import lax
from jax.experimental import pallas as pl
from jax.experimental.pallas import tpu as pltpu
from jax.experimental.pallas import tpu_sc as plsc

D_MODEL = 1024
SEQ_LEN = 2048
BATCH = 4
NUM_CORES = 2
NUM_SUBCORES = 16
NUM_WORKERS = NUM_CORES * NUM_SUBCORES
POS_PER_WORKER = SEQ_LEN // NUM_WORKERS
CHUNK_POS = 8
NUM_CHUNKS = POS_PER_WORKER // CHUNK_POS
ROWS_PER_CHUNK = BATCH * CHUNK_POS
LANES = 16


def _body(ids_hbm, wtab_hbm, ptab_hbm, out_hbm, idx_v, wbuf, pbuf,
          wsem0, wsem1, psem0, psem1, osem0, osem1):
    wsem = (wsem0, wsem1)
    psem = (psem0, psem1)
    osem = (osem0, osem1)
    wid = lax.axis_index("s") * NUM_CORES + lax.axis_index("c")
    p0 = wid * POS_PER_WORKER

    pltpu.sync_copy(
        ids_hbm.at[pl.ds(wid * BATCH * POS_PER_WORKER, BATCH * POS_PER_WORKER)],
        idx_v)

    gathers = [None] * NUM_CHUNKS
    poscopies = [None] * NUM_CHUNKS
    stores = [None] * NUM_CHUNKS

    def issue(t):
        par = t % 2
        gathers[t] = pltpu.async_copy(
            wtab_hbm.at[idx_v.at[pl.ds(t * ROWS_PER_CHUNK, ROWS_PER_CHUNK)]],
            wbuf.at[par], wsem[par])
        poscopies[t] = pltpu.async_copy(
            ptab_hbm.at[pl.ds(p0 + t * CHUNK_POS, CHUNK_POS)],
            pbuf.at[par], psem[par])

    issue(0)
    for t in range(NUM_CHUNKS):
        par = t % 2
        if t + 1 < NUM_CHUNKS:
            if t >= 1:
                for c in stores[t - 1]:
                    c.wait()
            issue(t + 1)
        gathers[t].wait()
        poscopies[t].wait()

        def add_pos(p, carry):
            @plsc.parallel_loop(0, D_MODEL, step=LANES, unroll=8)
            def add_vec(o):
                js = pl.ds(o, LANES)
                pv = pbuf[par, p, js]
                for b in range(BATCH):
                    wbuf[par, b * CHUNK_POS + p, js] = (
                        wbuf[par, b * CHUNK_POS + p, js] + pv)

            return carry

        lax.fori_loop(0, CHUNK_POS, add_pos, 0)

        stores[t] = [
            pltpu.async_copy(
                wbuf.at[par, pl.ds(b * CHUNK_POS, CHUNK_POS)],
                out_hbm.at[b, pl.ds(p0 + t * CHUNK_POS, CHUNK_POS), :],
                osem[par])
            for b in range(BATCH)
        ]

    for t in (NUM_CHUNKS - 2, NUM_CHUNKS - 1):
        for c in stores[t]:
            c.wait()


_embed = pl.kernel(
    _body,
    out_type=jax.ShapeDtypeStruct((BATCH, SEQ_LEN, D_MODEL), jnp.float32),
    mesh=plsc.VectorSubcoreMesh(core_axis_name="c", subcore_axis_name="s"),
    scratch_types=[
        pltpu.VMEM((NUM_WORKERS * 0 + BATCH * POS_PER_WORKER,), jnp.int32),
        pltpu.VMEM((2, ROWS_PER_CHUNK, D_MODEL), jnp.float32),
        pltpu.VMEM((2, CHUNK_POS, D_MODEL), jnp.float32),
        pltpu.SemaphoreType.DMA,
        pltpu.SemaphoreType.DMA,
        pltpu.SemaphoreType.DMA,
        pltpu.SemaphoreType.DMA,
        pltpu.SemaphoreType.DMA,
        pltpu.SemaphoreType.DMA,
    ],
)


@jax.jit
def kernel(input_ids, word_embedding, position_embedding):
    batch, seq = input_ids.shape
    ids = (input_ids.astype(jnp.int32)
           .reshape(BATCH, NUM_WORKERS, NUM_CHUNKS, CHUNK_POS)
           .transpose(1, 2, 0, 3)
           .reshape(-1))
    return _embed(ids, word_embedding, position_embedding)
